# Initial kernel scaffold; baseline (speedup 1.0000x reference)
import jax
import jax.numpy as jnp
from jax.experimental import pallas as pl


def kernel(points1, points2):
    raise NotImplementedError("write your pallas kernel here")


# speedup vs baseline: 1.5065x; 1.5065x over previous
import jax
import jax.numpy as jnp
from jax.experimental import pallas as pl

BI = 512


def _chamfer_block(x_ref, y_ref, d1_ref, d2_ref):
    i = pl.program_id(1)
    x = x_ref[0]
    y = y_ref[0]
    d = None
    for c in range(3):
        diff = x[:, c : c + 1] - y[c : c + 1, :]
        sq = diff * diff
        d = sq if d is None else d + sq
    d1_ref[0] = jnp.min(d, axis=1)[None, :]
    colmin = jnp.min(d, axis=0)[None, :]

    @pl.when(i == 0)
    def _init():
        d2_ref[0] = colmin

    @pl.when(i != 0)
    def _acc():
        d2_ref[0] = jnp.minimum(d2_ref[0], colmin)


def kernel(points1, points2):
    B, N, _ = points1.shape
    M = points2.shape[1]
    p2t = jnp.swapaxes(points2, 1, 2)

    grid = (B, N // BI)
    d1, d2 = pl.pallas_call(
        _chamfer_block,
        grid=grid,
        in_specs=[
            pl.BlockSpec((1, BI, 3), lambda b, i: (b, i, 0)),
            pl.BlockSpec((1, 3, M), lambda b, i: (b, 0, 0)),
        ],
        out_specs=[
            pl.BlockSpec((1, 1, BI), lambda b, i: (b, 0, i)),
            pl.BlockSpec((1, 1, M), lambda b, i: (b, 0, 0)),
        ],
        out_shape=[
            jax.ShapeDtypeStruct((B, 1, N), jnp.float32),
            jax.ShapeDtypeStruct((B, 1, M), jnp.float32),
        ],
    )(points1, p2t)

    cost = (jnp.mean(d1, axis=-1) + jnp.mean(d2, axis=-1)) * 0.5
    return jnp.sum(cost)


# speedup vs baseline: 1.5903x; 1.0556x over previous
import jax
import jax.numpy as jnp
from jax.experimental import pallas as pl

BI = 2048


def _chamfer_block(x_ref, y_ref, d1_ref, d2_ref):
    i = pl.program_id(1)
    x = x_ref[0]
    y = y_ref[0]
    d = None
    for c in range(3):
        diff = x[:, c : c + 1] - y[c : c + 1, :]
        sq = diff * diff
        d = sq if d is None else d + sq
    d1_ref[0] = jnp.min(d, axis=1)[None, :]
    colmin = jnp.min(d, axis=0)[None, :]

    @pl.when(i == 0)
    def _init():
        d2_ref[0] = colmin

    @pl.when(i != 0)
    def _acc():
        d2_ref[0] = jnp.minimum(d2_ref[0], colmin)


def kernel(points1, points2):
    B, N, _ = points1.shape
    M = points2.shape[1]
    p2t = jnp.swapaxes(points2, 1, 2)

    grid = (B, N // BI)
    d1, d2 = pl.pallas_call(
        _chamfer_block,
        grid=grid,
        in_specs=[
            pl.BlockSpec((1, BI, 3), lambda b, i: (b, i, 0)),
            pl.BlockSpec((1, 3, M), lambda b, i: (b, 0, 0)),
        ],
        out_specs=[
            pl.BlockSpec((1, 1, BI), lambda b, i: (b, 0, i)),
            pl.BlockSpec((1, 1, M), lambda b, i: (b, 0, 0)),
        ],
        out_shape=[
            jax.ShapeDtypeStruct((B, 1, N), jnp.float32),
            jax.ShapeDtypeStruct((B, 1, M), jnp.float32),
        ],
    )(points1, p2t)

    cost = (jnp.mean(d1, axis=-1) + jnp.mean(d2, axis=-1)) * 0.5
    return jnp.sum(cost)


# speedup vs baseline: 1.7658x; 1.1103x over previous
import jax
import jax.numpy as jnp
from jax.experimental import pallas as pl


def _chamfer_block(x_ref, y_ref, out_ref):
    b = pl.program_id(0)
    x = x_ref[0]
    y = y_ref[0]
    n = x.shape[0]
    m = y.shape[1]
    d = None
    for c in range(3):
        diff = x[:, c : c + 1] - y[c : c + 1, :]
        sq = diff * diff
        d = sq if d is None else d + sq
    rowmin = jnp.min(d, axis=1)
    colmin = jnp.min(d, axis=0)
    cost = jnp.sum(rowmin) * (0.5 / n) + jnp.sum(colmin) * (0.5 / m)

    @pl.when(b == 0)
    def _init():
        out_ref[...] = cost[None, None]

    @pl.when(b != 0)
    def _acc():
        out_ref[...] += cost[None, None]


def kernel(points1, points2):
    B, N, _ = points1.shape
    M = points2.shape[1]
    p2t = jnp.swapaxes(points2, 1, 2)

    out = pl.pallas_call(
        _chamfer_block,
        grid=(B,),
        in_specs=[
            pl.BlockSpec((1, N, 3), lambda b: (b, 0, 0)),
            pl.BlockSpec((1, 3, M), lambda b: (b, 0, 0)),
        ],
        out_specs=pl.BlockSpec((1, 1), lambda b: (0, 0)),
        out_shape=jax.ShapeDtypeStruct((1, 1), jnp.float32),
    )(points1, p2t)

    return out[0, 0]
